# baseline (speedup 1.0000x reference)
import functools

import jax
import jax.numpy as jnp
from jax import lax
from jax.experimental import pallas as pl
from jax.experimental.pallas import tpu as pltpu
from jax.experimental.pallas import tpu_sc as plsc

_N = 10000
_E = 320000
_NC = 2
_NS = 16
_NW = _NC * _NS
_CH = 128
_NCH = 80
_EPAD = _NW * _NCH * _CH
_NPAD = 10112
_ROWS_SUB = _NPAD // _NS
_D = 128

_HIGH = lax.Precision.HIGHEST


def _mesh():
    return plsc.VectorSubcoreMesh(core_axis_name="c", subcore_axis_name="s")


_DEGW = 16


def _sc_degree(dst_c, ones_blk, zeros_init):
    @functools.partial(
        pl.kernel,
        out_type=jax.ShapeDtypeStruct((_NC, _NPAD, _DEGW), jnp.float32),
        mesh=_mesh(),
        compiler_params=pltpu.CompilerParams(use_tc_tiling_on_sc=False),
        scratch_types=[
            pltpu.VMEM((_NCH, _CH), jnp.int32),
            pltpu.VMEM((_CH, _DEGW), jnp.float32),
            pltpu.VMEM_SHARED((_NPAD, _DEGW), jnp.float32),
        ],
    )
    def k(dst_hbm, ones_hbm, zero_hbm, out_hbm, dst_v, ones_v, acc_sh):
        c = lax.axis_index("c")
        s = lax.axis_index("s")
        wid = c * _NS + s
        r0 = s * _ROWS_SUB
        pltpu.sync_copy(zero_hbm.at[pl.ds(r0, _ROWS_SUB)],
                        acc_sh.at[pl.ds(r0, _ROWS_SUB)])
        pltpu.sync_copy(dst_hbm.at[wid], dst_v)
        pltpu.sync_copy(ones_hbm, ones_v)
        plsc.subcore_barrier()

        def body(j, carry):
            pltpu.sync_copy(ones_v, acc_sh.at[dst_v.at[j]], add=True)
            return carry

        lax.fori_loop(0, _NCH, body, 0)
        plsc.subcore_barrier()
        pltpu.sync_copy(acc_sh.at[pl.ds(r0, _ROWS_SUB)],
                        out_hbm.at[c].at[pl.ds(r0, _ROWS_SUB)])

    return k(dst_c, ones_blk, zeros_init)


def _sc_propagate(g, zeros_init, src_c, dst_c):
    @functools.partial(
        pl.kernel,
        out_type=jax.ShapeDtypeStruct((_NC, _NPAD, _D), jnp.float32),
        mesh=_mesh(),
        scratch_types=[
            pltpu.VMEM((_NCH, _CH), jnp.int32),
            pltpu.VMEM((_NCH, _CH), jnp.int32),
            pltpu.VMEM((_CH, _D), jnp.float32),
            pltpu.VMEM_SHARED((_NPAD, _D), jnp.float32),
        ],
    )
    def k(g_hbm, zero_hbm, src_hbm, dst_hbm, out_hbm, src_v, dst_v, buf, acc_sh):
        c = lax.axis_index("c")
        s = lax.axis_index("s")
        wid = c * _NS + s
        r0 = s * _ROWS_SUB
        pltpu.sync_copy(zero_hbm.at[pl.ds(r0, _ROWS_SUB)],
                        acc_sh.at[pl.ds(r0, _ROWS_SUB)])
        pltpu.sync_copy(src_hbm.at[wid], src_v)
        pltpu.sync_copy(dst_hbm.at[wid], dst_v)
        plsc.subcore_barrier()

        def body(j, carry):
            pltpu.sync_copy(g_hbm.at[src_v.at[j]], buf)
            pltpu.sync_copy(buf, acc_sh.at[dst_v.at[j]], add=True)
            return carry

        lax.fori_loop(0, _NCH, body, 0)
        plsc.subcore_barrier()
        pltpu.sync_copy(acc_sh.at[pl.ds(r0, _ROWS_SUB)],
                        out_hbm.at[c].at[pl.ds(r0, _ROWS_SUB)])

    return k(g, zeros_init, src_c, dst_c)


def _sc_propagate_narrow(g, zeros_init, src_c, dst_c, d):
    @functools.partial(
        pl.kernel,
        out_type=jax.ShapeDtypeStruct((_NC, _NPAD, d), jnp.float32),
        mesh=_mesh(),
        compiler_params=pltpu.CompilerParams(use_tc_tiling_on_sc=False),
        scratch_types=[
            pltpu.VMEM((_NCH, _CH), jnp.int32),
            pltpu.VMEM((_NCH, _CH), jnp.int32),
            pltpu.VMEM((_CH, d), jnp.float32),
            pltpu.VMEM_SHARED((_NPAD, d), jnp.float32),
        ],
    )
    def k(g_hbm, zero_hbm, src_hbm, dst_hbm, out_hbm, src_v, dst_v, buf,
          acc_sh):
        c = lax.axis_index("c")
        s = lax.axis_index("s")
        wid = c * _NS + s
        r0 = s * _ROWS_SUB
        pltpu.sync_copy(zero_hbm.at[pl.ds(r0, _ROWS_SUB)],
                        acc_sh.at[pl.ds(r0, _ROWS_SUB)])
        pltpu.sync_copy(src_hbm.at[wid], src_v)
        pltpu.sync_copy(dst_hbm.at[wid], dst_v)
        plsc.subcore_barrier()

        def body(j, carry):
            pltpu.sync_copy(g_hbm.at[src_v.at[j]], buf)
            pltpu.sync_copy(buf, acc_sh.at[dst_v.at[j]], add=True)
            return carry

        lax.fori_loop(0, _NCH, body, 0)
        plsc.subcore_barrier()
        pltpu.sync_copy(acc_sh.at[pl.ds(r0, _ROWS_SUB)],
                        out_hbm.at[c].at[pl.ds(r0, _ROWS_SUB)])

    return k(g, zeros_init, src_c, dst_c)


def _sc_edge_gather(p, q, src_c, dst_c):
    @functools.partial(
        pl.kernel,
        out_type=[jax.ShapeDtypeStruct((_EPAD, _D), jnp.bfloat16),
                  jax.ShapeDtypeStruct((_EPAD, _D), jnp.bfloat16)],
        mesh=_mesh(),
        compiler_params=pltpu.CompilerParams(use_tc_tiling_on_sc=False),
        scratch_types=[
            pltpu.VMEM((_NCH, _CH), jnp.int32),
            pltpu.VMEM((_NCH, _CH), jnp.int32),
            pltpu.VMEM((_CH, _D), jnp.bfloat16),
            pltpu.VMEM((_CH, _D), jnp.bfloat16),
            pltpu.VMEM((_CH, _D), jnp.bfloat16),
            pltpu.VMEM((_CH, _D), jnp.bfloat16),
            pltpu.SemaphoreType.DMA, pltpu.SemaphoreType.DMA,
            pltpu.SemaphoreType.DMA, pltpu.SemaphoreType.DMA,
        ],
    )
    def k(p_hbm, q_hbm, src_hbm, dst_hbm, r1_hbm, r2_hbm, src_v, dst_v,
          p0, p1, q0, q1, wp0, wp1, wq0, wq1):
        pb = (p0, p1)
        qb = (q0, q1)
        wp = (wp0, wp1)
        wq = (wq0, wq1)
        c = lax.axis_index("c")
        s = lax.axis_index("s")
        wid = c * _NS + s
        base = wid * _NCH * _CH
        pltpu.sync_copy(src_hbm.at[wid], src_v)
        pltpu.sync_copy(dst_hbm.at[wid], dst_v)

        def gathers(j, b):
            pltpu.sync_copy(p_hbm.at[src_v.at[j]], pb[b])
            pltpu.sync_copy(q_hbm.at[dst_v.at[j]], qb[b])

        def fire_writes(j, b):
            row = base + j * _CH
            pltpu.async_copy(pb[b], r1_hbm.at[pl.ds(row, _CH)], wp[b])
            pltpu.async_copy(qb[b], r2_hbm.at[pl.ds(row, _CH)], wq[b])

        def wait_writes(b):
            pltpu.make_async_copy(pb[b], r1_hbm.at[pl.ds(0, _CH)], wp[b]).wait()
            pltpu.make_async_copy(qb[b], r2_hbm.at[pl.ds(0, _CH)], wq[b]).wait()

        for j in (0, 1):
            gathers(j, j)
            fire_writes(j, j)

        @pl.loop(2, _NCH, step=2)
        def duos(i):
            for b in range(2):
                j = i + b
                wait_writes(b)
                gathers(j, b)
                fire_writes(j, b)

        wait_writes(0)
        wait_writes(1)

    return k(p, q, src_c, dst_c)


def _tc1_body(deg_ref, x_ref, w_ref, dinv_ref, g_ref):
    deg = deg_ref[0, :, 0:1] + deg_ref[1, :, 0:1] + 1.0
    dinv = lax.rsqrt(deg)
    dinv_ref[...] = dinv
    g_ref[...] = dinv * jnp.dot(x_ref[...], w_ref[...], precision=_HIGH)


def _tc1(deg, xp, w1p):
    return pl.pallas_call(
        _tc1_body,
        out_shape=[jax.ShapeDtypeStruct((_NPAD, 1), jnp.float32),
                   jax.ShapeDtypeStruct((_NPAD, w1p.shape[1]), jnp.float32)],
    )(deg, xp, w1p)


def _tc_mid_body(acc_ref, g_ref, dinv_ref, b_ref, w_ref, out_ref):
    dinv = dinv_ref[...]
    h = jnp.maximum(dinv * (acc_ref[0] + acc_ref[1] + g_ref[...]) + b_ref[...], 0.0)
    out_ref[...] = dinv * jnp.dot(h, w_ref[...], precision=_HIGH)


def _tc_mid(acc, g, dinv, b2d, w):
    return pl.pallas_call(
        _tc_mid_body,
        out_shape=jax.ShapeDtypeStruct((_NPAD, w.shape[1]), jnp.float32),
    )(acc, g, dinv, b2d, w)


def _tc_fin_body(acc_ref, g_ref, dinv_ref, b_ref, m1a_ref, m1b_ref, mb1_ref,
                 p_ref, q_ref):
    dinv = dinv_ref[...]
    h = jnp.maximum(dinv * (acc_ref[0] + acc_ref[1] + g_ref[...]) + b_ref[...], 0.0)
    p_ref[...] = (jnp.dot(h, m1a_ref[...], precision=_HIGH)
                  + mb1_ref[...]).astype(jnp.bfloat16)
    q_ref[...] = jnp.dot(h, m1b_ref[...], precision=_HIGH).astype(jnp.bfloat16)


def _tc_fin(acc, g, dinv, b2d, m1a, m1b, mb1_2d):
    return pl.pallas_call(
        _tc_fin_body,
        out_shape=[jax.ShapeDtypeStruct((_NPAD, _D), jnp.bfloat16),
                   jax.ShapeDtypeStruct((_NPAD, _D), jnp.bfloat16)],
    )(acc, g, dinv, b2d, m1a, m1b, mb1_2d)


_MLP_BLK = 2048


def _tc_mlp_body(r1_ref, r2_ref, m2_ref, mb2_ref, m3_ref, mb3_ref, m4_ref,
                 mb4_ref, m5_ref, mb5_ref, o_ref):
    ef = jnp.maximum(r1_ref[...].astype(jnp.float32)
                     + r2_ref[...].astype(jnp.float32), 0.0)
    ef = jnp.maximum(jnp.dot(ef, m2_ref[...], precision=_HIGH) + mb2_ref[...], 0.0)
    ef = jnp.maximum(jnp.dot(ef, m3_ref[...], precision=_HIGH) + mb3_ref[...], 0.0)
    ef = jnp.maximum(jnp.dot(ef, m4_ref[...], precision=_HIGH) + mb4_ref[...], 0.0)
    ef = jnp.dot(ef, m5_ref[...], precision=_HIGH) + mb5_ref[...]
    o_ref[...] = jax.nn.sigmoid(ef)


def _tc_mlp(r1, r2, m2, mb2, m3, mb3, m4, mb4, m5, mb5):
    n_blk = _EPAD // _MLP_BLK
    blk = lambda c: pl.BlockSpec((_MLP_BLK, c), lambda i: (i, 0))
    full = lambda a: pl.BlockSpec(a.shape, lambda i: tuple(0 for _ in a.shape))
    return pl.pallas_call(
        _tc_mlp_body,
        grid=(n_blk,),
        in_specs=[blk(_D), blk(_D),
                  full(m2), full(mb2), full(m3), full(mb3),
                  full(m4), full(mb4), full(m5), full(mb5)],
        out_specs=pl.BlockSpec((_MLP_BLK, 1), lambda i: (i, 0)),
        out_shape=jax.ShapeDtypeStruct((_EPAD, 1), jnp.float32),
    )(r1, r2, m2, mb2, m3, mb3, m4, mb4, m5, mb5)


def kernel(x, edge_index, W1, b1, W2, b2, W3, b3, W4, b4, W5, b5,
           M1, mb1, M2, mb2, M3, mb3, M4, mb4, M5, mb5):
    f32 = jnp.float32
    src = edge_index[0].astype(jnp.int32)
    dst = edge_index[1].astype(jnp.int32)
    src_c = jnp.pad(src, (0, _EPAD - _E), constant_values=_N).reshape(_NW, _NCH, _CH)
    dst_c = jnp.pad(dst, (0, _EPAD - _E), constant_values=_N).reshape(_NW, _NCH, _CH)
    xp = jnp.pad(x, ((0, _NPAD - _N), (0, 0)))

    zeros_nd = jnp.zeros((_NPAD, _D), f32)
    ones_blk = jnp.ones((_CH, _DEGW), f32)
    deg = _sc_degree(dst_c, ones_blk, jnp.zeros((_NPAD, _DEGW), f32))
    w1p = jnp.pad(W1, ((0, 0), (0, 8)))
    b1p = jnp.pad(b1, (0, 8))
    w2p = jnp.pad(W2, ((0, 8), (0, 0)))
    dinv, g = _tc1(deg, xp, w1p)

    for b, w in ((b1p, w2p), (b2, W3), (b3, W4), (b4, W5)):
        d_in = w.shape[0]
        acc = _sc_propagate_narrow(g, jnp.zeros((_NPAD, d_in), f32),
                                   src_c, dst_c, d_in)
        g = _tc_mid(acc, g, dinv, b.reshape(1, -1), w)

    acc5 = _sc_propagate(g, zeros_nd, src_c, dst_c)
    p, q = _tc_fin(acc5, g, dinv, b5.reshape(1, -1), M1[:_D], M1[_D:],
                   mb1.reshape(1, -1))

    r1, r2 = _sc_edge_gather(p, q, src_c, dst_c)
    out = _tc_mlp(r1, r2, M2, mb2.reshape(1, -1), M3, mb3.reshape(1, -1),
                  M4, mb4.reshape(1, -1), M5, mb5.reshape(1, -1))
    return out[:_E]


# speedup vs baseline: 1.2317x; 1.2317x over previous
import functools

import jax
import jax.numpy as jnp
from jax import lax
from jax.experimental import pallas as pl
from jax.experimental.pallas import tpu as pltpu
from jax.experimental.pallas import tpu_sc as plsc

_N = 10000
_E = 320000
_NC = 2
_NS = 16
_NW = _NC * _NS
_CH = 128
_NCH = 80
_EPAD = _NW * _NCH * _CH
_NPAD = 10112
_ROWS_SUB = _NPAD // _NS
_D = 128

_HIGH = lax.Precision.HIGHEST


def _mesh():
    return plsc.VectorSubcoreMesh(core_axis_name="c", subcore_axis_name="s")


_DEGW = 16


def _sc_degree(dst_c, ones_blk, zeros_init):
    @functools.partial(
        pl.kernel,
        out_type=jax.ShapeDtypeStruct((_NC, _NPAD, _DEGW), jnp.float32),
        mesh=_mesh(),
        compiler_params=pltpu.CompilerParams(use_tc_tiling_on_sc=False),
        scratch_types=[
            pltpu.VMEM((_NCH, _CH), jnp.int32),
            pltpu.VMEM((_CH, _DEGW), jnp.float32),
            pltpu.VMEM_SHARED((_NPAD, _DEGW), jnp.float32),
        ],
    )
    def k(dst_hbm, ones_hbm, zero_hbm, out_hbm, dst_v, ones_v, acc_sh):
        c = lax.axis_index("c")
        s = lax.axis_index("s")
        wid = c * _NS + s
        r0 = s * _ROWS_SUB
        pltpu.sync_copy(zero_hbm.at[pl.ds(r0, _ROWS_SUB)],
                        acc_sh.at[pl.ds(r0, _ROWS_SUB)])
        pltpu.sync_copy(dst_hbm.at[wid], dst_v)
        pltpu.sync_copy(ones_hbm, ones_v)
        plsc.subcore_barrier()

        def body(j, carry):
            pltpu.sync_copy(ones_v, acc_sh.at[dst_v.at[j]], add=True)
            return carry

        lax.fori_loop(0, _NCH, body, 0)
        plsc.subcore_barrier()
        pltpu.sync_copy(acc_sh.at[pl.ds(r0, _ROWS_SUB)],
                        out_hbm.at[c].at[pl.ds(r0, _ROWS_SUB)])

    return k(dst_c, ones_blk, zeros_init)


def _sc_propagate(g, zeros_init, src_c, dst_c):
    @functools.partial(
        pl.kernel,
        out_type=jax.ShapeDtypeStruct((_NC, _NPAD, _D), jnp.float32),
        mesh=_mesh(),
        scratch_types=[
            pltpu.VMEM((_NCH, _CH), jnp.int32),
            pltpu.VMEM((_NCH, _CH), jnp.int32),
            pltpu.VMEM((_CH, _D), jnp.float32),
            pltpu.VMEM_SHARED((_NPAD, _D), jnp.float32),
        ],
    )
    def k(g_hbm, zero_hbm, src_hbm, dst_hbm, out_hbm, src_v, dst_v, buf, acc_sh):
        c = lax.axis_index("c")
        s = lax.axis_index("s")
        wid = c * _NS + s
        r0 = s * _ROWS_SUB
        pltpu.sync_copy(zero_hbm.at[pl.ds(r0, _ROWS_SUB)],
                        acc_sh.at[pl.ds(r0, _ROWS_SUB)])
        pltpu.sync_copy(src_hbm.at[wid], src_v)
        pltpu.sync_copy(dst_hbm.at[wid], dst_v)
        plsc.subcore_barrier()

        def body(j, carry):
            pltpu.sync_copy(g_hbm.at[src_v.at[j]], buf)
            pltpu.sync_copy(buf, acc_sh.at[dst_v.at[j]], add=True)
            return carry

        lax.fori_loop(0, _NCH, body, 0)
        plsc.subcore_barrier()
        pltpu.sync_copy(acc_sh.at[pl.ds(r0, _ROWS_SUB)],
                        out_hbm.at[c].at[pl.ds(r0, _ROWS_SUB)])

    return k(g, zeros_init, src_c, dst_c)


def _sc_propagate_narrow(g, zeros_init, src_c, dst_c, d):
    @functools.partial(
        pl.kernel,
        out_type=jax.ShapeDtypeStruct((_NC, _NPAD, d), jnp.float32),
        mesh=_mesh(),
        compiler_params=pltpu.CompilerParams(use_tc_tiling_on_sc=False),
        scratch_types=[
            pltpu.VMEM((_NCH, _CH), jnp.int32),
            pltpu.VMEM((_NCH, _CH), jnp.int32),
            pltpu.VMEM((_CH, d), jnp.float32),
            pltpu.VMEM_SHARED((_NPAD, d), jnp.float32),
        ],
    )
    def k(g_hbm, zero_hbm, src_hbm, dst_hbm, out_hbm, src_v, dst_v, buf,
          acc_sh):
        c = lax.axis_index("c")
        s = lax.axis_index("s")
        wid = c * _NS + s
        r0 = s * _ROWS_SUB
        pltpu.sync_copy(zero_hbm.at[pl.ds(r0, _ROWS_SUB)],
                        acc_sh.at[pl.ds(r0, _ROWS_SUB)])
        pltpu.sync_copy(src_hbm.at[wid], src_v)
        pltpu.sync_copy(dst_hbm.at[wid], dst_v)
        plsc.subcore_barrier()

        def body(j, carry):
            pltpu.sync_copy(g_hbm.at[src_v.at[j]], buf)
            pltpu.sync_copy(buf, acc_sh.at[dst_v.at[j]], add=True)
            return carry

        lax.fori_loop(0, _NCH, body, 0)
        plsc.subcore_barrier()
        pltpu.sync_copy(acc_sh.at[pl.ds(r0, _ROWS_SUB)],
                        out_hbm.at[c].at[pl.ds(r0, _ROWS_SUB)])

    return k(g, zeros_init, src_c, dst_c)


def _sc_edge_gather(p, q, src_c, dst_c):
    @functools.partial(
        pl.kernel,
        out_type=[jax.ShapeDtypeStruct((_EPAD, _D), jnp.float32),
                  jax.ShapeDtypeStruct((_EPAD, _D), jnp.float32)],
        mesh=_mesh(),
        scratch_types=[
            pltpu.VMEM((_NCH, _CH), jnp.int32),
            pltpu.VMEM((_NCH, _CH), jnp.int32),
            pltpu.VMEM((_CH, _D), jnp.float32),
            pltpu.VMEM((_CH, _D), jnp.float32),
            pltpu.VMEM((_CH, _D), jnp.float32),
            pltpu.VMEM((_CH, _D), jnp.float32),
            pltpu.SemaphoreType.DMA, pltpu.SemaphoreType.DMA,
            pltpu.SemaphoreType.DMA, pltpu.SemaphoreType.DMA,
            pltpu.SemaphoreType.DMA, pltpu.SemaphoreType.DMA,
            pltpu.SemaphoreType.DMA, pltpu.SemaphoreType.DMA,
        ],
    )
    def k(p_hbm, q_hbm, src_hbm, dst_hbm, r1_hbm, r2_hbm, src_v, dst_v,
          p0, p1, q0, q1, gp0, gp1, gq0, gq1, wp0, wp1, wq0, wq1):
        pb = (p0, p1)
        qb = (q0, q1)
        gp = (gp0, gp1)
        gq = (gq0, gq1)
        wp = (wp0, wp1)
        wq = (wq0, wq1)
        c = lax.axis_index("c")
        s = lax.axis_index("s")
        wid = c * _NS + s
        base = wid * _NCH * _CH
        pltpu.sync_copy(src_hbm.at[wid], src_v)
        pltpu.sync_copy(dst_hbm.at[wid], dst_v)

        def fire_gathers(j, b):
            pltpu.async_copy(p_hbm.at[src_v.at[j]], pb[b], gp[b])
            pltpu.async_copy(q_hbm.at[dst_v.at[j]], qb[b], gq[b])

        def wait_gathers(b):
            pltpu.make_async_copy(p_hbm.at[pl.ds(0, _CH)], pb[b], gp[b]).wait()
            pltpu.make_async_copy(q_hbm.at[pl.ds(0, _CH)], qb[b], gq[b]).wait()

        def fire_writes(j, b):
            row = base + j * _CH
            pltpu.async_copy(pb[b], r1_hbm.at[pl.ds(row, _CH)], wp[b])
            pltpu.async_copy(qb[b], r2_hbm.at[pl.ds(row, _CH)], wq[b])

        def wait_writes(b):
            pltpu.make_async_copy(pb[b], r1_hbm.at[pl.ds(0, _CH)], wp[b]).wait()
            pltpu.make_async_copy(qb[b], r2_hbm.at[pl.ds(0, _CH)], wq[b]).wait()

        fire_gathers(0, 0)
        fire_gathers(1, 1)
        wait_gathers(0)
        fire_writes(0, 0)

        @pl.loop(1, _NCH - 1, step=2)
        def pairs(i):
            for bb in range(2):
                j = i + bb
                b = (1 + bb) % 2
                wait_writes(bb)
                fire_gathers(j + 1, bb)
                wait_gathers(b)
                fire_writes(j, b)

        wait_gathers((_NCH - 1) % 2)
        fire_writes(_NCH - 1, (_NCH - 1) % 2)
        wait_writes(0)
        wait_writes(1)

    return k(p, q, src_c, dst_c)


def _tc1_body(deg_ref, x_ref, w_ref, dinv_ref, g_ref):
    deg = deg_ref[0, :, 0:1] + deg_ref[1, :, 0:1] + 1.0
    dinv = lax.rsqrt(deg)
    dinv_ref[...] = dinv
    g_ref[...] = dinv * jnp.dot(x_ref[...], w_ref[...], precision=_HIGH)


def _tc1(deg, xp, w1p):
    return pl.pallas_call(
        _tc1_body,
        out_shape=[jax.ShapeDtypeStruct((_NPAD, 1), jnp.float32),
                   jax.ShapeDtypeStruct((_NPAD, w1p.shape[1]), jnp.float32)],
    )(deg, xp, w1p)


def _tc_mid_body(acc_ref, g_ref, dinv_ref, b_ref, w_ref, out_ref):
    dinv = dinv_ref[...]
    h = jnp.maximum(dinv * (acc_ref[0] + acc_ref[1] + g_ref[...]) + b_ref[...], 0.0)
    out_ref[...] = dinv * jnp.dot(h, w_ref[...], precision=_HIGH)


def _tc_mid(acc, g, dinv, b2d, w):
    return pl.pallas_call(
        _tc_mid_body,
        out_shape=jax.ShapeDtypeStruct((_NPAD, w.shape[1]), jnp.float32),
    )(acc, g, dinv, b2d, w)


def _tc_fin_body(acc_ref, g_ref, dinv_ref, b_ref, m1a_ref, m1b_ref, mb1_ref,
                 p_ref, q_ref):
    dinv = dinv_ref[...]
    h = jnp.maximum(dinv * (acc_ref[0] + acc_ref[1] + g_ref[...]) + b_ref[...], 0.0)
    p_ref[...] = jnp.dot(h, m1a_ref[...], precision=_HIGH) + mb1_ref[...]
    q_ref[...] = jnp.dot(h, m1b_ref[...], precision=_HIGH)


def _tc_fin(acc, g, dinv, b2d, m1a, m1b, mb1_2d):
    return pl.pallas_call(
        _tc_fin_body,
        out_shape=[jax.ShapeDtypeStruct((_NPAD, _D), jnp.float32),
                   jax.ShapeDtypeStruct((_NPAD, _D), jnp.float32)],
    )(acc, g, dinv, b2d, m1a, m1b, mb1_2d)


_MLP_BLK = 2048


def _tc_mlp_body(r1_ref, r2_ref, m2_ref, mb2_ref, m3_ref, mb3_ref, m4_ref,
                 mb4_ref, m5_ref, mb5_ref, o_ref):
    ef = jnp.maximum(r1_ref[...] + r2_ref[...], 0.0)
    ef = jnp.maximum(jnp.dot(ef, m2_ref[...], precision=_HIGH) + mb2_ref[...], 0.0)
    ef = jnp.maximum(jnp.dot(ef, m3_ref[...], precision=_HIGH) + mb3_ref[...], 0.0)
    ef = jnp.maximum(jnp.dot(ef, m4_ref[...], precision=_HIGH) + mb4_ref[...], 0.0)
    ef = jnp.dot(ef, m5_ref[...], precision=_HIGH) + mb5_ref[...]
    o_ref[...] = jax.nn.sigmoid(ef)


def _tc_mlp(r1, r2, m2, mb2, m3, mb3, m4, mb4, m5, mb5):
    n_blk = _EPAD // _MLP_BLK
    blk = lambda c: pl.BlockSpec((_MLP_BLK, c), lambda i: (i, 0))
    full = lambda a: pl.BlockSpec(a.shape, lambda i: tuple(0 for _ in a.shape))
    return pl.pallas_call(
        _tc_mlp_body,
        grid=(n_blk,),
        in_specs=[blk(_D), blk(_D),
                  full(m2), full(mb2), full(m3), full(mb3),
                  full(m4), full(mb4), full(m5), full(mb5)],
        out_specs=pl.BlockSpec((_MLP_BLK, 1), lambda i: (i, 0)),
        out_shape=jax.ShapeDtypeStruct((_EPAD, 1), jnp.float32),
    )(r1, r2, m2, mb2, m3, mb3, m4, mb4, m5, mb5)


def kernel(x, edge_index, W1, b1, W2, b2, W3, b3, W4, b4, W5, b5,
           M1, mb1, M2, mb2, M3, mb3, M4, mb4, M5, mb5):
    f32 = jnp.float32
    src = edge_index[0].astype(jnp.int32)
    dst = edge_index[1].astype(jnp.int32)
    src_c = jnp.pad(src, (0, _EPAD - _E), constant_values=_N).reshape(_NW, _NCH, _CH)
    dst_c = jnp.pad(dst, (0, _EPAD - _E), constant_values=_N).reshape(_NW, _NCH, _CH)
    xp = jnp.pad(x, ((0, _NPAD - _N), (0, 0)))

    zeros_nd = jnp.zeros((_NPAD, _D), f32)
    ones_blk = jnp.ones((_CH, _DEGW), f32)
    deg = _sc_degree(dst_c, ones_blk, jnp.zeros((_NPAD, _DEGW), f32))
    w1p = jnp.pad(W1, ((0, 0), (0, 8)))
    b1p = jnp.pad(b1, (0, 8))
    w2p = jnp.pad(W2, ((0, 8), (0, 0)))
    dinv, g = _tc1(deg, xp, w1p)

    for b, w in ((b1p, w2p), (b2, W3), (b3, W4), (b4, W5)):
        d_in = w.shape[0]
        acc = _sc_propagate_narrow(g, jnp.zeros((_NPAD, d_in), f32),
                                   src_c, dst_c, d_in)
        g = _tc_mid(acc, g, dinv, b.reshape(1, -1), w)

    acc5 = _sc_propagate(g, zeros_nd, src_c, dst_c)
    p, q = _tc_fin(acc5, g, dinv, b5.reshape(1, -1), M1[:_D], M1[_D:],
                   mb1.reshape(1, -1))

    r1, r2 = _sc_edge_gather(p, q, src_c, dst_c)
    out = _tc_mlp(r1, r2, M2, mb2.reshape(1, -1), M3, mb3.reshape(1, -1),
                  M4, mb4.reshape(1, -1), M5, mb5.reshape(1, -1))
    return out[:_E]


# speedup vs baseline: 1.3543x; 1.0996x over previous
import functools

import jax
import jax.numpy as jnp
from jax import lax
from jax.experimental import pallas as pl
from jax.experimental.pallas import tpu as pltpu
from jax.experimental.pallas import tpu_sc as plsc

_N = 10000
_E = 320000
_NC = 2
_NS = 16
_NW = _NC * _NS
_CH = 128
_NCH = 80
_EPAD = _NW * _NCH * _CH
_NPAD = 10112
_ROWS_SUB = _NPAD // _NS
_D = 128

_HIGH = lax.Precision.HIGHEST


def _mesh():
    return plsc.VectorSubcoreMesh(core_axis_name="c", subcore_axis_name="s")


_DEGW = 16


def _sc_degree(dst_c, ones_blk, zeros_init):
    @functools.partial(
        pl.kernel,
        out_type=jax.ShapeDtypeStruct((_NC, _NPAD, _DEGW), jnp.float32),
        mesh=_mesh(),
        compiler_params=pltpu.CompilerParams(use_tc_tiling_on_sc=False),
        scratch_types=[
            pltpu.VMEM((_NCH, _CH), jnp.int32),
            pltpu.VMEM((_CH, _DEGW), jnp.float32),
            pltpu.VMEM_SHARED((_NPAD, _DEGW), jnp.float32),
        ],
    )
    def k(dst_hbm, ones_hbm, zero_hbm, out_hbm, dst_v, ones_v, acc_sh):
        c = lax.axis_index("c")
        s = lax.axis_index("s")
        wid = c * _NS + s
        r0 = s * _ROWS_SUB
        pltpu.sync_copy(zero_hbm.at[pl.ds(r0, _ROWS_SUB)],
                        acc_sh.at[pl.ds(r0, _ROWS_SUB)])
        pltpu.sync_copy(dst_hbm.at[wid], dst_v)
        pltpu.sync_copy(ones_hbm, ones_v)
        plsc.subcore_barrier()

        def body(j, carry):
            pltpu.sync_copy(ones_v, acc_sh.at[dst_v.at[j]], add=True)
            return carry

        lax.fori_loop(0, _NCH, body, 0)
        plsc.subcore_barrier()
        pltpu.sync_copy(acc_sh.at[pl.ds(r0, _ROWS_SUB)],
                        out_hbm.at[c].at[pl.ds(r0, _ROWS_SUB)])

    return k(dst_c, ones_blk, zeros_init)


def _sc_propagate(g, zeros_init, src_c, dst_c):
    @functools.partial(
        pl.kernel,
        out_type=jax.ShapeDtypeStruct((_NC, _NPAD, _D), jnp.float32),
        mesh=_mesh(),
        scratch_types=[
            pltpu.VMEM((_NCH, _CH), jnp.int32),
            pltpu.VMEM((_NCH, _CH), jnp.int32),
            pltpu.VMEM((_CH, _D), jnp.float32),
            pltpu.VMEM_SHARED((_NPAD, _D), jnp.float32),
        ],
    )
    def k(g_hbm, zero_hbm, src_hbm, dst_hbm, out_hbm, src_v, dst_v, buf, acc_sh):
        c = lax.axis_index("c")
        s = lax.axis_index("s")
        wid = c * _NS + s
        r0 = s * _ROWS_SUB
        pltpu.sync_copy(zero_hbm.at[pl.ds(r0, _ROWS_SUB)],
                        acc_sh.at[pl.ds(r0, _ROWS_SUB)])
        pltpu.sync_copy(src_hbm.at[wid], src_v)
        pltpu.sync_copy(dst_hbm.at[wid], dst_v)
        plsc.subcore_barrier()

        def body(j, carry):
            pltpu.sync_copy(g_hbm.at[src_v.at[j]], buf)
            pltpu.sync_copy(buf, acc_sh.at[dst_v.at[j]], add=True)
            return carry

        lax.fori_loop(0, _NCH, body, 0)
        plsc.subcore_barrier()
        pltpu.sync_copy(acc_sh.at[pl.ds(r0, _ROWS_SUB)],
                        out_hbm.at[c].at[pl.ds(r0, _ROWS_SUB)])

    return k(g, zeros_init, src_c, dst_c)


def _sc_propagate_narrow(g, zeros_init, src_c, dst_c, d):
    @functools.partial(
        pl.kernel,
        out_type=jax.ShapeDtypeStruct((_NC, _NPAD, d), jnp.float32),
        mesh=_mesh(),
        compiler_params=pltpu.CompilerParams(use_tc_tiling_on_sc=False),
        scratch_types=[
            pltpu.VMEM((_NCH, _CH), jnp.int32),
            pltpu.VMEM((_NCH, _CH), jnp.int32),
            pltpu.VMEM((_CH, d), jnp.float32),
            pltpu.VMEM_SHARED((_NPAD, d), jnp.float32),
        ],
    )
    def k(g_hbm, zero_hbm, src_hbm, dst_hbm, out_hbm, src_v, dst_v, buf,
          acc_sh):
        c = lax.axis_index("c")
        s = lax.axis_index("s")
        wid = c * _NS + s
        r0 = s * _ROWS_SUB
        pltpu.sync_copy(zero_hbm.at[pl.ds(r0, _ROWS_SUB)],
                        acc_sh.at[pl.ds(r0, _ROWS_SUB)])
        pltpu.sync_copy(src_hbm.at[wid], src_v)
        pltpu.sync_copy(dst_hbm.at[wid], dst_v)
        plsc.subcore_barrier()

        def body(j, carry):
            pltpu.sync_copy(g_hbm.at[src_v.at[j]], buf)
            pltpu.sync_copy(buf, acc_sh.at[dst_v.at[j]], add=True)
            return carry

        lax.fori_loop(0, _NCH, body, 0)
        plsc.subcore_barrier()
        pltpu.sync_copy(acc_sh.at[pl.ds(r0, _ROWS_SUB)],
                        out_hbm.at[c].at[pl.ds(r0, _ROWS_SUB)])

    return k(g, zeros_init, src_c, dst_c)


def _sc_edge_gather(p, q, src_c, dst_c, nch):
    ne = _NW * nch * _CH
    @functools.partial(
        pl.kernel,
        out_type=[jax.ShapeDtypeStruct((ne, _D), jnp.float32),
                  jax.ShapeDtypeStruct((ne, _D), jnp.float32)],
        mesh=_mesh(),
        scratch_types=[
            pltpu.VMEM((nch, _CH), jnp.int32),
            pltpu.VMEM((nch, _CH), jnp.int32),
            pltpu.VMEM((_CH, _D), jnp.float32),
            pltpu.VMEM((_CH, _D), jnp.float32),
            pltpu.VMEM((_CH, _D), jnp.float32),
            pltpu.VMEM((_CH, _D), jnp.float32),
            pltpu.SemaphoreType.DMA, pltpu.SemaphoreType.DMA,
            pltpu.SemaphoreType.DMA, pltpu.SemaphoreType.DMA,
            pltpu.SemaphoreType.DMA, pltpu.SemaphoreType.DMA,
            pltpu.SemaphoreType.DMA, pltpu.SemaphoreType.DMA,
        ],
    )
    def k(p_hbm, q_hbm, src_hbm, dst_hbm, r1_hbm, r2_hbm, src_v, dst_v,
          p0, p1, q0, q1, gp0, gp1, gq0, gq1, wp0, wp1, wq0, wq1):
        pb = (p0, p1)
        qb = (q0, q1)
        gp = (gp0, gp1)
        gq = (gq0, gq1)
        wp = (wp0, wp1)
        wq = (wq0, wq1)
        c = lax.axis_index("c")
        s = lax.axis_index("s")
        wid = c * _NS + s
        base = wid * nch * _CH
        pltpu.sync_copy(src_hbm.at[wid], src_v)
        pltpu.sync_copy(dst_hbm.at[wid], dst_v)

        def fire_gathers(j, b):
            pltpu.async_copy(p_hbm.at[src_v.at[j]], pb[b], gp[b])
            pltpu.async_copy(q_hbm.at[dst_v.at[j]], qb[b], gq[b])

        def wait_gathers(b):
            pltpu.make_async_copy(p_hbm.at[pl.ds(0, _CH)], pb[b], gp[b]).wait()
            pltpu.make_async_copy(q_hbm.at[pl.ds(0, _CH)], qb[b], gq[b]).wait()

        def fire_writes(j, b):
            row = base + j * _CH
            pltpu.async_copy(pb[b], r1_hbm.at[pl.ds(row, _CH)], wp[b])
            pltpu.async_copy(qb[b], r2_hbm.at[pl.ds(row, _CH)], wq[b])

        def wait_writes(b):
            pltpu.make_async_copy(pb[b], r1_hbm.at[pl.ds(0, _CH)], wp[b]).wait()
            pltpu.make_async_copy(qb[b], r2_hbm.at[pl.ds(0, _CH)], wq[b]).wait()

        fire_gathers(0, 0)
        fire_gathers(1, 1)
        wait_gathers(0)
        fire_writes(0, 0)

        @pl.loop(1, nch - 1, step=2)
        def pairs(i):
            for bb in range(2):
                j = i + bb
                b = (1 + bb) % 2
                wait_writes(bb)
                fire_gathers(j + 1, bb)
                wait_gathers(b)
                fire_writes(j, b)

        wait_gathers((nch - 1) % 2)
        fire_writes(nch - 1, (nch - 1) % 2)
        wait_writes(0)
        wait_writes(1)

    return k(p, q, src_c, dst_c)


def _tc_x1_body(x_ref, w_ref, h_ref):
    h_ref[...] = jnp.dot(x_ref[...], w_ref[...], precision=_HIGH)


def _tc_x1(xp, w1p):
    return pl.pallas_call(
        _tc_x1_body,
        out_shape=jax.ShapeDtypeStruct((_NPAD, w1p.shape[1]), jnp.float32),
    )(xp, w1p)


def _tc1_body(deg_ref, h_ref, dinv_ref, g_ref):
    deg = deg_ref[0, :, 0:1] + deg_ref[1, :, 0:1] + 1.0
    dinv = lax.rsqrt(deg)
    dinv_ref[...] = dinv
    g_ref[...] = dinv * h_ref[...]


def _tc1(deg, h1):
    return pl.pallas_call(
        _tc1_body,
        out_shape=[jax.ShapeDtypeStruct((_NPAD, 1), jnp.float32),
                   jax.ShapeDtypeStruct((_NPAD, h1.shape[1]), jnp.float32)],
    )(deg, h1)


def _tc_mid_body(acc_ref, g_ref, dinv_ref, b_ref, w_ref, out_ref):
    dinv = dinv_ref[...]
    h = jnp.maximum(dinv * (acc_ref[0] + acc_ref[1] + g_ref[...]) + b_ref[...], 0.0)
    out_ref[...] = dinv * jnp.dot(h, w_ref[...], precision=_HIGH)


def _tc_mid(acc, g, dinv, b2d, w):
    return pl.pallas_call(
        _tc_mid_body,
        out_shape=jax.ShapeDtypeStruct((_NPAD, w.shape[1]), jnp.float32),
    )(acc, g, dinv, b2d, w)


def _tc_fin_body(acc_ref, g_ref, dinv_ref, b_ref, m1a_ref, m1b_ref, mb1_ref,
                 p_ref, q_ref):
    dinv = dinv_ref[...]
    h = jnp.maximum(dinv * (acc_ref[0] + acc_ref[1] + g_ref[...]) + b_ref[...], 0.0)
    p_ref[...] = jnp.dot(h, m1a_ref[...], precision=_HIGH) + mb1_ref[...]
    q_ref[...] = jnp.dot(h, m1b_ref[...], precision=_HIGH)


def _tc_fin(acc, g, dinv, b2d, m1a, m1b, mb1_2d):
    return pl.pallas_call(
        _tc_fin_body,
        out_shape=[jax.ShapeDtypeStruct((_NPAD, _D), jnp.float32),
                   jax.ShapeDtypeStruct((_NPAD, _D), jnp.float32)],
    )(acc, g, dinv, b2d, m1a, m1b, mb1_2d)


_MLP_BLK = 2048


def _tc_mlp_body(r1_ref, r2_ref, m2_ref, mb2_ref, m3_ref, mb3_ref, m4_ref,
                 mb4_ref, m5_ref, mb5_ref, o_ref):
    ef = jnp.maximum(r1_ref[...] + r2_ref[...], 0.0)
    ef = jnp.maximum(jnp.dot(ef, m2_ref[...], precision=_HIGH) + mb2_ref[...], 0.0)
    ef = jnp.maximum(jnp.dot(ef, m3_ref[...], precision=_HIGH) + mb3_ref[...], 0.0)
    ef = jnp.maximum(jnp.dot(ef, m4_ref[...], precision=_HIGH) + mb4_ref[...], 0.0)
    ef = jnp.dot(ef, m5_ref[...], precision=_HIGH) + mb5_ref[...]
    o_ref[...] = jax.nn.sigmoid(ef)


def _tc_mlp(r1, r2, m2, mb2, m3, mb3, m4, mb4, m5, mb5):
    n_blk = r1.shape[0] // _MLP_BLK
    blk = lambda c: pl.BlockSpec((_MLP_BLK, c), lambda i: (i, 0))
    full = lambda a: pl.BlockSpec(a.shape, lambda i: tuple(0 for _ in a.shape))
    return pl.pallas_call(
        _tc_mlp_body,
        grid=(n_blk,),
        in_specs=[blk(_D), blk(_D),
                  full(m2), full(mb2), full(m3), full(mb3),
                  full(m4), full(mb4), full(m5), full(mb5)],
        out_specs=pl.BlockSpec((_MLP_BLK, 1), lambda i: (i, 0)),
        out_shape=jax.ShapeDtypeStruct((r1.shape[0], 1), jnp.float32),
    )(r1, r2, m2, mb2, m3, mb3, m4, mb4, m5, mb5)


def kernel(x, edge_index, W1, b1, W2, b2, W3, b3, W4, b4, W5, b5,
           M1, mb1, M2, mb2, M3, mb3, M4, mb4, M5, mb5):
    f32 = jnp.float32
    src = edge_index[0].astype(jnp.int32)
    dst = edge_index[1].astype(jnp.int32)
    src_c = jnp.pad(src, (0, _EPAD - _E), constant_values=_N).reshape(_NW, _NCH, _CH)
    dst_c = jnp.pad(dst, (0, _EPAD - _E), constant_values=_N).reshape(_NW, _NCH, _CH)
    xp = jnp.pad(x, ((0, _NPAD - _N), (0, 0)))

    zeros_nd = jnp.zeros((_NPAD, _D), f32)
    ones_blk = jnp.ones((_CH, _DEGW), f32)
    h1 = _tc_x1(xp, jnp.pad(W1, ((0, 0), (0, 8))))
    deg = _sc_degree(dst_c, ones_blk, jnp.zeros((_NPAD, _DEGW), f32))
    w1p = jnp.pad(W1, ((0, 0), (0, 8)))
    b1p = jnp.pad(b1, (0, 8))
    w2p = jnp.pad(W2, ((0, 8), (0, 0)))
    dinv, g = _tc1(deg, h1)

    for b, w in ((b1p, w2p), (b2, W3), (b3, W4), (b4, W5)):
        d_in = w.shape[0]
        acc = _sc_propagate_narrow(g, jnp.zeros((_NPAD, d_in), f32),
                                   src_c, dst_c, d_in)
        g = _tc_mid(acc, g, dinv, b.reshape(1, -1), w)

    acc5 = _sc_propagate(g, zeros_nd, src_c, dst_c)
    p, q = _tc_fin(acc5, g, dinv, b5.reshape(1, -1), M1[:_D], M1[_D:],
                   mb1.reshape(1, -1))

    half = _NCH // 2
    mlp_w = (M2, mb2.reshape(1, -1), M3, mb3.reshape(1, -1),
             M4, mb4.reshape(1, -1), M5, mb5.reshape(1, -1))
    outs = []
    for lo in (0, half):
        r1, r2 = _sc_edge_gather(p, q, src_c[:, lo:lo + half],
                                 dst_c[:, lo:lo + half], half)
        outs.append(_tc_mlp(r1, r2, *mlp_w).reshape(_NW, half * _CH))
    out = jnp.concatenate(outs, axis=1).reshape(_EPAD, 1)
    return out[:_E]
